# baseline (speedup 1.0000x reference)
import jax
import jax.numpy as jnp
import numpy as np
from jax import lax
from jax.experimental import pallas as pl
from jax.experimental.pallas import tpu as pltpu

_B = 8
_N = 1024
_IN = 256
_HID = 128
_K = 10
_TAU = 0.5
_ROWS = 256
_NT = _N // _ROWS
_NTILES = _B * _NT
_PAIR = 2
_NSLOT = 7
_DIST = 6
_CLIP = float(-np.log(np.float32(1e-8)))
_CLIP2 = float(2.0 * np.log2(np.e) * _CLIP)
_NEG2LOG2E = float(-2.0 * np.log2(np.e))


def _body(adj_ref, x_ref, w1pi_ref, w1mu_ref, w1sig_ref,
          b1pi_ref, b1mu_ref, b1sig_ref,
          w2pi_ref, w2mu_ref, w2sig_ref,
          b2pi_ref, b2mu_ref, b2sig_ref, upi_ref, noise_ref,
          u_hbm, o_ref, u0b, u1b, srow_sc, scol_sc, adjb_sc, w1_sc,
          w2_sc, b1_sc, b2_sc, sems):
    f32 = jnp.float32
    bf = jnp.bfloat16
    a = pl.program_id(0)
    i = pl.program_id(1)
    nt_pair = _PAIR * _NT

    def _start(slot, bb, tt):
        hr = _ROWS // 2
        for h in range(2):
            pltpu.make_async_copy(
                u_hbm.at[bb, pl.ds(tt * _ROWS + h * hr, hr), 0, :],
                u0b.at[slot, pl.ds(h * hr, hr)], sems.at[slot, 0]).start()
            pltpu.make_async_copy(
                u_hbm.at[bb, pl.ds(tt * _ROWS + h * hr, hr), 1, :],
                u1b.at[slot, pl.ds(h * hr, hr)], sems.at[slot, 1]).start()

    @pl.when(i == 0)
    def _phase_a():
        @pl.when(a == 0)
        def _prime():
            for t in range(_DIST):
                _start(t % _NSLOT, t // _NT, t % _NT)
            adjb_sc[...] = adj_ref[...].astype(bf)
            w1_sc[:, 0:_HID] = w1pi_ref[...].astype(bf)
            w1_sc[:, _HID:2 * _HID] = w1mu_ref[...].astype(bf)
            w1_sc[:, 2 * _HID:3 * _HID] = w1sig_ref[...].astype(bf)
            w2_sc[...] = jnp.zeros((3 * _HID, 32), bf)
            w2_sc[0:_HID, 0:_K] = w2pi_ref[...].astype(bf)
            w2_sc[_HID:2 * _HID, _K:2 * _K] = w2mu_ref[...].astype(bf)
            w2_sc[2 * _HID:3 * _HID, 2 * _K:3 * _K] = w2sig_ref[...].astype(bf)
            b1_sc[0:1, 0:_HID] = b1pi_ref[...].reshape(1, _HID)
            b1_sc[0:1, _HID:2 * _HID] = b1mu_ref[...].reshape(1, _HID)
            b1_sc[0:1, 2 * _HID:3 * _HID] = b1sig_ref[...].reshape(1, _HID)
            b2_sc[...] = jnp.zeros((1, 32), f32)
            b2_sc[0:1, 0:_K] = b2pi_ref[...].reshape(1, _K)
            b2_sc[0:1, _K:2 * _K] = b2mu_ref[...].reshape(1, _K)
            b2_sc[0:1, 2 * _K:3 * _K] = b2sig_ref[...].reshape(1, _K)

        lane = lax.broadcasted_iota(jnp.int32, (_N, 32), 1)
        adjb = adjb_sc[...]
        for p in range(_PAIR):
            xb = x_ref[p].astype(bf)
            ax = jnp.dot(adjb, xb,
                         preferred_element_type=f32).astype(bf)
            h = jnp.maximum(jnp.dot(ax, w1_sc[...],
                                    preferred_element_type=f32)
                            + b1_sc[...], 0.0)
            ah = jnp.dot(adjb, h.astype(bf),
                         preferred_element_type=f32)
            out = jnp.dot(ah.astype(bf), w2_sc[...],
                          preferred_element_type=f32) + b2_sc[...]
            slab = jnp.concatenate(
                [upi_ref[pl.ds(_PAIR * a + p + _B * k, 1), :]
                 for k in range(_K)], axis=0)
            g = jnp.transpose(-jnp.log(-jnp.log(slab)))
            gpad = jnp.concatenate(
                [g, jnp.zeros((_N, 32 - _K), f32)], axis=1)
            z = jnp.where(lane < _K, (out + gpad) * f32(1.0 / _TAU), -1e30)
            z = z - jnp.max(z, axis=1, keepdims=True)
            e = jnp.exp(z)
            y = e / jnp.sum(e, axis=1, keepdims=True)
            nrow = jnp.transpose(
                noise_ref[pl.ds(_PAIR * a + p, 1), :])
            w = (jnp.roll(y, _K, axis=1)
                 + nrow * jnp.roll(y, 2 * _K, axis=1))
            s_val = jnp.sum(out * w, axis=1, keepdims=True)
            srow_sc[p] = s_val * f32(_NEG2LOG2E)
            scol_sc[p] = jnp.transpose(s_val)

    @pl.when(i > 0)
    def _phase_b():
        t = a * nt_pair + (i - 1)
        slot = lax.rem(t, _NSLOT)

        @pl.when(t + _DIST < _NTILES)
        def _prefetch():
            t2 = t + _DIST
            _start(lax.rem(t2, _NSLOT), lax.div(t2, _NT), lax.rem(t2, _NT))

        bb = lax.div(t, _NT)
        tt = lax.rem(t, _NT)
        pltpu.make_async_copy(
            u_hbm.at[bb, pl.ds(tt * _ROWS, _ROWS), 0, :],
            u0b.at[slot], sems.at[slot, 0]).wait()
        pltpu.make_async_copy(
            u_hbm.at[bb, pl.ds(tt * _ROWS, _ROWS), 1, :],
            u1b.at[slot], sems.at[slot, 1]).wait()

        p = lax.div(i - 1, _NT)
        l0 = jnp.log2(u0b[slot])
        l1 = jnp.log2(u1b[slot])
        srow = srow_sc[p, pl.ds(tt * _ROWS, _ROWS), :]
        e2 = lax.exp2(jnp.clip(srow * scol_sc[p], -_CLIP2, _CLIP2))
        n0 = l0 * l0 * e2
        n1 = l1 * l1
        o_ref[0] = n1 / (n1 + n0)


def kernel(x, adj_t, mu_W1, mu_b1, mu_W2, mu_b2, sig_W1, sig_b1, sig_W2,
           sig_b2, pi_W1, pi_b1, pi_W2, pi_b2, norm_noise, u_pi, u_pp):
    f32 = jnp.float32
    upi = jnp.transpose(u_pi, (2, 0, 1)).reshape(_K * _B, _N)
    upt = jnp.transpose(u_pp, (0, 1, 3, 2))

    const = lambda *idx: (lambda a, i: idx)
    a = pl.pallas_call(
        _body,
        grid=(_B // _PAIR, 1 + _PAIR * _NT),
        in_specs=[
            pl.BlockSpec((_N, _N), const(0, 0)),
            pl.BlockSpec((_PAIR, _N, _IN), lambda a, i: (a, 0, 0)),
            pl.BlockSpec((_IN, _HID), const(0, 0)),
            pl.BlockSpec((_IN, _HID), const(0, 0)),
            pl.BlockSpec((_IN, _HID), const(0, 0)),
            pl.BlockSpec((_HID,), const(0)),
            pl.BlockSpec((_HID,), const(0)),
            pl.BlockSpec((_HID,), const(0)),
            pl.BlockSpec((_HID, _K), const(0, 0)),
            pl.BlockSpec((_HID, _K), const(0, 0)),
            pl.BlockSpec((_HID, _K), const(0, 0)),
            pl.BlockSpec((_K,), const(0)),
            pl.BlockSpec((_K,), const(0)),
            pl.BlockSpec((_K,), const(0)),
            pl.BlockSpec((_K * _B, _N), const(0, 0)),
            pl.BlockSpec((_B, _N), const(0, 0)),
            pl.BlockSpec(memory_space=pltpu.MemorySpace.HBM),
        ],
        out_specs=pl.BlockSpec(
            (1, _ROWS, _N),
            lambda a, i: (_PAIR * a + jnp.maximum(i - 1, 0) // _NT,
                          jnp.maximum(i - 1, 0) % _NT, 0)),
        out_shape=jax.ShapeDtypeStruct((_B, _N, _N), f32),
        scratch_shapes=[
            pltpu.VMEM((_NSLOT, _ROWS, _N), f32),
            pltpu.VMEM((_NSLOT, _ROWS, _N), f32),
            pltpu.VMEM((_PAIR, _N, 1), f32),
            pltpu.VMEM((_PAIR, 1, _N), f32),
            pltpu.VMEM((_N, _N), jnp.bfloat16),
            pltpu.VMEM((_IN, 3 * _HID), jnp.bfloat16),
            pltpu.VMEM((3 * _HID, 32), jnp.bfloat16),
            pltpu.VMEM((1, 3 * _HID), f32),
            pltpu.VMEM((1, 32), f32),
            pltpu.SemaphoreType.DMA((_NSLOT, 2)),
        ],
    )(adj_t, x, pi_W1, mu_W1, sig_W1, pi_b1, mu_b1, sig_b1,
      pi_W2, mu_W2, sig_W2, pi_b2, mu_b2, sig_b2, upi, norm_noise, upt)
    return a


# speedup vs baseline: 1.0275x; 1.0275x over previous
import jax
import jax.numpy as jnp
import numpy as np
from jax import lax
from jax.experimental import pallas as pl
from jax.experimental.pallas import tpu as pltpu

_B = 8
_N = 1024
_IN = 256
_HID = 128
_K = 10
_TAU = 0.5
_ROWS = 512
_NT = _N // _ROWS
_NTILES = _B * _NT
_PAIR = 2
_NSLOT = 5
_DIST = 4
_CLIP = float(-np.log(np.float32(1e-8)))
_CLIP2 = float(2.0 * np.log2(np.e) * _CLIP)
_NEG2LOG2E = float(-2.0 * np.log2(np.e))


def _body(adj_ref, x_ref, w1pi_ref, w1mu_ref, w1sig_ref,
          b1pi_ref, b1mu_ref, b1sig_ref,
          w2pi_ref, w2mu_ref, w2sig_ref,
          b2pi_ref, b2mu_ref, b2sig_ref, upi_ref, noise_ref,
          u_hbm, o_ref, u0b, u1b, srow_sc, scol_sc, adjb_sc, w1_sc,
          w2_sc, b1_sc, b2_sc, sems):
    f32 = jnp.float32
    bf = jnp.bfloat16
    a = pl.program_id(0)
    i = pl.program_id(1)
    nt_pair = _PAIR * _NT

    def _start(slot, bb, tt):
        hr = _ROWS // 2
        for h in range(2):
            pltpu.make_async_copy(
                u_hbm.at[bb, pl.ds(tt * _ROWS + h * hr, hr), 0, :],
                u0b.at[slot, pl.ds(h * hr, hr)], sems.at[slot, 0]).start()
            pltpu.make_async_copy(
                u_hbm.at[bb, pl.ds(tt * _ROWS + h * hr, hr), 1, :],
                u1b.at[slot, pl.ds(h * hr, hr)], sems.at[slot, 1]).start()

    @pl.when(i == 0)
    def _phase_a():
        @pl.when(a == 0)
        def _prime():
            for t in range(_DIST):
                _start(t % _NSLOT, t // _NT, t % _NT)
            adjb_sc[...] = adj_ref[...].astype(bf)
            w1_sc[:, 0:_HID] = w1pi_ref[...].astype(bf)
            w1_sc[:, _HID:2 * _HID] = w1mu_ref[...].astype(bf)
            w1_sc[:, 2 * _HID:3 * _HID] = w1sig_ref[...].astype(bf)
            w2_sc[...] = jnp.zeros((3 * _HID, 32), bf)
            w2_sc[0:_HID, 0:_K] = w2pi_ref[...].astype(bf)
            w2_sc[_HID:2 * _HID, _K:2 * _K] = w2mu_ref[...].astype(bf)
            w2_sc[2 * _HID:3 * _HID, 2 * _K:3 * _K] = w2sig_ref[...].astype(bf)
            b1_sc[0:1, 0:_HID] = b1pi_ref[...].reshape(1, _HID)
            b1_sc[0:1, _HID:2 * _HID] = b1mu_ref[...].reshape(1, _HID)
            b1_sc[0:1, 2 * _HID:3 * _HID] = b1sig_ref[...].reshape(1, _HID)
            b2_sc[...] = jnp.zeros((1, 32), f32)
            b2_sc[0:1, 0:_K] = b2pi_ref[...].reshape(1, _K)
            b2_sc[0:1, _K:2 * _K] = b2mu_ref[...].reshape(1, _K)
            b2_sc[0:1, 2 * _K:3 * _K] = b2sig_ref[...].reshape(1, _K)

        lane = lax.broadcasted_iota(jnp.int32, (_N, 32), 1)
        adjb = adjb_sc[...]
        for p in range(_PAIR):
            xb = x_ref[p].astype(bf)
            ax = jnp.dot(adjb, xb,
                         preferred_element_type=f32).astype(bf)
            h = jnp.maximum(jnp.dot(ax, w1_sc[...],
                                    preferred_element_type=f32)
                            + b1_sc[...], 0.0)
            ah = jnp.dot(adjb, h.astype(bf),
                         preferred_element_type=f32)
            out = jnp.dot(ah.astype(bf), w2_sc[...],
                          preferred_element_type=f32) + b2_sc[...]
            slab = jnp.concatenate(
                [upi_ref[pl.ds(_PAIR * a + p + _B * k, 1), :]
                 for k in range(_K)], axis=0)
            g = jnp.transpose(-jnp.log(-jnp.log(slab)))
            gpad = jnp.concatenate(
                [g, jnp.zeros((_N, 32 - _K), f32)], axis=1)
            z = jnp.where(lane < _K, (out + gpad) * f32(1.0 / _TAU), -1e30)
            z = z - jnp.max(z, axis=1, keepdims=True)
            e = jnp.exp(z)
            y = e / jnp.sum(e, axis=1, keepdims=True)
            nrow = jnp.transpose(
                noise_ref[pl.ds(_PAIR * a + p, 1), :])
            w = (jnp.roll(y, _K, axis=1)
                 + nrow * jnp.roll(y, 2 * _K, axis=1))
            s_val = jnp.sum(out * w, axis=1, keepdims=True)
            srow_sc[p] = s_val * f32(_NEG2LOG2E)
            scol_sc[p] = jnp.transpose(s_val)

    @pl.when(i > 0)
    def _phase_b():
        t = a * nt_pair + (i - 1)
        slot = lax.rem(t, _NSLOT)

        @pl.when(t + _DIST < _NTILES)
        def _prefetch():
            t2 = t + _DIST
            _start(lax.rem(t2, _NSLOT), lax.div(t2, _NT), lax.rem(t2, _NT))

        bb = lax.div(t, _NT)
        tt = lax.rem(t, _NT)
        pltpu.make_async_copy(
            u_hbm.at[bb, pl.ds(tt * _ROWS, _ROWS), 0, :],
            u0b.at[slot], sems.at[slot, 0]).wait()
        pltpu.make_async_copy(
            u_hbm.at[bb, pl.ds(tt * _ROWS, _ROWS), 1, :],
            u1b.at[slot], sems.at[slot, 1]).wait()

        p = lax.div(i - 1, _NT)
        l0 = jnp.log2(u0b[slot])
        l1 = jnp.log2(u1b[slot])
        srow = srow_sc[p, pl.ds(tt * _ROWS, _ROWS), :]
        e2 = lax.exp2(jnp.clip(srow * scol_sc[p], -_CLIP2, _CLIP2))
        n0 = l0 * l0 * e2
        n1 = l1 * l1
        o_ref[0] = n1 / (n1 + n0)


def kernel(x, adj_t, mu_W1, mu_b1, mu_W2, mu_b2, sig_W1, sig_b1, sig_W2,
           sig_b2, pi_W1, pi_b1, pi_W2, pi_b2, norm_noise, u_pi, u_pp):
    f32 = jnp.float32
    upi = jnp.transpose(u_pi, (2, 0, 1)).reshape(_K * _B, _N)
    upt = jnp.transpose(u_pp, (0, 1, 3, 2))

    const = lambda *idx: (lambda a, i: idx)
    a = pl.pallas_call(
        _body,
        grid=(_B // _PAIR, 1 + _PAIR * _NT),
        in_specs=[
            pl.BlockSpec((_N, _N), const(0, 0)),
            pl.BlockSpec((_PAIR, _N, _IN), lambda a, i: (a, 0, 0)),
            pl.BlockSpec((_IN, _HID), const(0, 0)),
            pl.BlockSpec((_IN, _HID), const(0, 0)),
            pl.BlockSpec((_IN, _HID), const(0, 0)),
            pl.BlockSpec((_HID,), const(0)),
            pl.BlockSpec((_HID,), const(0)),
            pl.BlockSpec((_HID,), const(0)),
            pl.BlockSpec((_HID, _K), const(0, 0)),
            pl.BlockSpec((_HID, _K), const(0, 0)),
            pl.BlockSpec((_HID, _K), const(0, 0)),
            pl.BlockSpec((_K,), const(0)),
            pl.BlockSpec((_K,), const(0)),
            pl.BlockSpec((_K,), const(0)),
            pl.BlockSpec((_K * _B, _N), const(0, 0)),
            pl.BlockSpec((_B, _N), const(0, 0)),
            pl.BlockSpec(memory_space=pltpu.MemorySpace.HBM),
        ],
        out_specs=pl.BlockSpec(
            (1, _ROWS, _N),
            lambda a, i: (_PAIR * a + jnp.maximum(i - 1, 0) // _NT,
                          jnp.maximum(i - 1, 0) % _NT, 0)),
        out_shape=jax.ShapeDtypeStruct((_B, _N, _N), f32),
        scratch_shapes=[
            pltpu.VMEM((_NSLOT, _ROWS, _N), f32),
            pltpu.VMEM((_NSLOT, _ROWS, _N), f32),
            pltpu.VMEM((_PAIR, _N, 1), f32),
            pltpu.VMEM((_PAIR, 1, _N), f32),
            pltpu.VMEM((_N, _N), jnp.bfloat16),
            pltpu.VMEM((_IN, 3 * _HID), jnp.bfloat16),
            pltpu.VMEM((3 * _HID, 32), jnp.bfloat16),
            pltpu.VMEM((1, 3 * _HID), f32),
            pltpu.VMEM((1, 32), f32),
            pltpu.SemaphoreType.DMA((_NSLOT, 2)),
        ],
    )(adj_t, x, pi_W1, mu_W1, sig_W1, pi_b1, mu_b1, sig_b1,
      pi_W2, mu_W2, sig_W2, pi_b2, mu_b2, sig_b2, upi, norm_noise, upt)
    return a
